# baseline (device time: 13843 ns/iter reference)
import jax
import jax.numpy as jnp
from jax import lax
from jax.experimental import pallas as pl
from jax.experimental.pallas import tpu as pltpu

N_DEV = 4


def kernel(x, w_mat):
    m_g, kb = x.shape
    k_g, n = w_mat.shape
    mb = m_g // N_DEV

    def body(x_hbm, w_hbm, out_hbm, x_ref, w_ref, ov_ref, sb_ref, xg_ref,
             send_sems, recv_sems, x_sem, w_sem, o_sem):
        my = lax.axis_index("i")

        x_cp = pltpu.make_async_copy(x_hbm, x_ref, x_sem)
        x_cp.start()
        w_cp = pltpu.make_async_copy(w_hbm, w_ref, w_sem)
        w_cp.start()

        barrier_sem = pltpu.get_barrier_semaphore()
        for o in range(1, N_DEV):
            peer = (my + o) % N_DEV
            pl.semaphore_signal(
                barrier_sem, inc=1,
                device_id=(peer,), device_id_type=pl.DeviceIdType.MESH,
            )
        pl.semaphore_wait(barrier_sem, N_DEV - 1)

        x_cp.wait()
        sends = []
        for o in (2, 1, 3):
            dst = (my + o) % N_DEV
            sb_ref[dst] = x_ref[pl.ds(dst * mb, mb), :].astype(jnp.bfloat16)
            rdma = pltpu.make_async_remote_copy(
                src_ref=sb_ref.at[dst],
                dst_ref=xg_ref.at[my],
                send_sem=send_sems.at[o - 1],
                recv_sem=recv_sems.at[my],
                device_id=(dst,),
                device_id_type=pl.DeviceIdType.MESH,
            )
            rdma.start()
            sends.append(rdma)

        w_cp.wait()
        acc = jnp.dot(
            x_ref[pl.ds(my * mb, mb), :],
            w_ref[pl.ds(my * kb, kb), :],
            preferred_element_type=jnp.float32,
        )

        for o in (1, 3, 2):
            src = (my + o) % N_DEV
            recv = pltpu.make_async_remote_copy(
                src_ref=sb_ref.at[src],
                dst_ref=xg_ref.at[src],
                send_sem=send_sems.at[o - 1],
                recv_sem=recv_sems.at[src],
                device_id=(src,),
                device_id_type=pl.DeviceIdType.MESH,
            )
            recv.wait_recv()
            acc = acc + jnp.dot(
                xg_ref[src].astype(jnp.float32),
                w_ref[pl.ds(src * kb, kb), :],
                preferred_element_type=jnp.float32,
            )

        c = 0.7978845608028654
        ov_ref[:, :] = 0.5 * acc * (1.0 + jnp.tanh(c * (acc + 0.044715 * acc * acc * acc)))
        out_cp = pltpu.make_async_copy(ov_ref, out_hbm, o_sem)
        out_cp.start()

        for rdma in sends:
            rdma.wait_send()
        out_cp.wait()

    return pl.pallas_call(
        body,
        out_shape=jax.ShapeDtypeStruct((mb, n), jnp.float32),
        in_specs=[
            pl.BlockSpec(memory_space=pl.ANY),
            pl.BlockSpec(memory_space=pl.ANY),
        ],
        out_specs=pl.BlockSpec(memory_space=pl.ANY),
        scratch_shapes=[
            pltpu.VMEM((m_g, kb), jnp.float32),
            pltpu.VMEM((k_g, n), jnp.float32),
            pltpu.VMEM((mb, n), jnp.float32),
            pltpu.VMEM((N_DEV, mb, kb), jnp.bfloat16),
            pltpu.VMEM((N_DEV, mb, kb), jnp.bfloat16),
            pltpu.SemaphoreType.DMA((N_DEV - 1,)),
            pltpu.SemaphoreType.DMA((N_DEV,)),
            pltpu.SemaphoreType.DMA,
            pltpu.SemaphoreType.DMA,
            pltpu.SemaphoreType.DMA,
        ],
        compiler_params=pltpu.CompilerParams(collective_id=0),
    )(x, w_mat)


# device time: 13823 ns/iter; 1.0014x vs baseline; 1.0014x over previous
import jax
import jax.numpy as jnp
from jax import lax
from jax.experimental import pallas as pl
from jax.experimental.pallas import tpu as pltpu

N_DEV = 4


def kernel(x, w_mat):
    m_g, kb = x.shape
    k_g, n = w_mat.shape
    mb = m_g // N_DEV

    def body(x_hbm, w_hbm, out_hbm, x_ref, w_ref, ov_ref, sb_ref, xg_ref,
             send_sems, recv_sems, x_sem, w_sem, o_sem):
        my = lax.axis_index("i")

        x_cp = pltpu.make_async_copy(x_hbm, x_ref, x_sem)
        x_cp.start()
        w_cp = pltpu.make_async_copy(w_hbm, w_ref, w_sem)
        w_cp.start()

        barrier_sem = pltpu.get_barrier_semaphore()
        for o in range(1, N_DEV):
            peer = (my + o) % N_DEV
            pl.semaphore_signal(
                barrier_sem, inc=1,
                device_id=(peer,), device_id_type=pl.DeviceIdType.MESH,
            )
        pl.semaphore_wait(barrier_sem, N_DEV - 1)

        x_cp.wait()
        sends = []
        for o in (2, 1, 3):
            dst = (my + o) % N_DEV
            sb_ref[dst] = x_ref[pl.ds(dst * mb, mb), :].astype(jnp.bfloat16)
            rdma = pltpu.make_async_remote_copy(
                src_ref=sb_ref.at[dst],
                dst_ref=xg_ref.at[my],
                send_sem=send_sems.at[o - 1],
                recv_sem=recv_sems.at[my],
                device_id=(dst,),
                device_id_type=pl.DeviceIdType.MESH,
            )
            rdma.start()
            sends.append(rdma)

        w_cp.wait()
        acc = jnp.dot(
            x_ref[pl.ds(my * mb, mb), :],
            w_ref[pl.ds(my * kb, kb), :],
            preferred_element_type=jnp.float32,
        )

        for o in (1, 3, 2):
            src = (my + o) % N_DEV
            recv = pltpu.make_async_remote_copy(
                src_ref=sb_ref.at[src],
                dst_ref=xg_ref.at[src],
                send_sem=send_sems.at[o - 1],
                recv_sem=recv_sems.at[src],
                device_id=(src,),
                device_id_type=pl.DeviceIdType.MESH,
            )
            recv.wait_recv()
            acc = acc + jnp.dot(
                xg_ref[src].astype(jnp.float32),
                w_ref[pl.ds(src * kb, kb), :],
                preferred_element_type=jnp.float32,
            )

        c = 0.7978845608028654
        ov_ref[:, :] = 0.5 * acc * (1.0 + jnp.tanh(c * (acc + 0.044715 * acc * acc * acc)))
        out_cp = pltpu.make_async_copy(ov_ref, out_hbm, o_sem)
        out_cp.start()

        for rdma in sends:
            rdma.wait_send()
        out_cp.wait()

    return pl.pallas_call(
        body,
        out_shape=jax.ShapeDtypeStruct((mb, n), jnp.float32),
        in_specs=[
            pl.BlockSpec(memory_space=pltpu.MemorySpace.HBM),
            pl.BlockSpec(memory_space=pltpu.MemorySpace.HBM),
        ],
        out_specs=pl.BlockSpec(memory_space=pltpu.MemorySpace.HBM),
        scratch_shapes=[
            pltpu.VMEM((m_g, kb), jnp.float32),
            pltpu.VMEM((k_g, n), jnp.float32),
            pltpu.VMEM((mb, n), jnp.float32),
            pltpu.VMEM((N_DEV, mb, kb), jnp.bfloat16),
            pltpu.VMEM((N_DEV, mb, kb), jnp.bfloat16),
            pltpu.SemaphoreType.DMA((N_DEV - 1,)),
            pltpu.SemaphoreType.DMA((N_DEV,)),
            pltpu.SemaphoreType.DMA,
            pltpu.SemaphoreType.DMA,
            pltpu.SemaphoreType.DMA,
        ],
        compiler_params=pltpu.CompilerParams(collective_id=0),
    )(x, w_mat)


# device time: 12177 ns/iter; 1.1368x vs baseline; 1.1352x over previous
import jax
import jax.numpy as jnp
from jax import lax
from jax.experimental import pallas as pl
from jax.experimental.pallas import tpu as pltpu

N_DEV = 4


def kernel(x, w_mat):
    m_g, kb = x.shape
    k_g, n = w_mat.shape
    mb = m_g // N_DEV
    nh = n // 2

    def body(x_ref, w_ref, out_ref, sb_ref, xg_ref, send_sems, recv_sems):
        my = lax.axis_index("i")

        for o in (2, 1, 3):
            dst = (my + o) % N_DEV
            sb_ref[dst] = x_ref[pl.ds(dst * mb, mb), :].astype(jnp.bfloat16)

        barrier_sem = pltpu.get_barrier_semaphore()
        for o in range(1, N_DEV):
            peer = (my + o) % N_DEV
            pl.semaphore_signal(
                barrier_sem, inc=1,
                device_id=(peer,), device_id_type=pl.DeviceIdType.MESH,
            )
        pl.semaphore_wait(barrier_sem, N_DEV - 1)

        sends = []
        for o in (2, 1, 3):
            dst = (my + o) % N_DEV
            rdma = pltpu.make_async_remote_copy(
                src_ref=sb_ref.at[dst],
                dst_ref=xg_ref.at[my],
                send_sem=send_sems.at[o - 1],
                recv_sem=recv_sems.at[my],
                device_id=(dst,),
                device_id_type=pl.DeviceIdType.MESH,
            )
            rdma.start()
            sends.append(rdma)

        xl = x_ref[pl.ds(my * mb, mb), :]
        wl = w_ref[pl.ds(my * kb, kb), :]
        acc0 = jnp.dot(xl, wl[:, :nh], preferred_element_type=jnp.float32)
        acc1 = jnp.dot(xl, wl[:, nh:], preferred_element_type=jnp.float32)

        for o in (1, 3, 2):
            src = (my + o) % N_DEV
            recv = pltpu.make_async_remote_copy(
                src_ref=sb_ref.at[src],
                dst_ref=xg_ref.at[src],
                send_sem=send_sems.at[o - 1],
                recv_sem=recv_sems.at[src],
                device_id=(src,),
                device_id_type=pl.DeviceIdType.MESH,
            )
            recv.wait_recv()
            xr = xg_ref[src].astype(jnp.float32)
            wr = w_ref[pl.ds(src * kb, kb), :]
            acc0 = acc0 + jnp.dot(xr, wr[:, :nh], preferred_element_type=jnp.float32)
            acc1 = acc1 + jnp.dot(xr, wr[:, nh:], preferred_element_type=jnp.float32)

        c = 0.7978845608028654
        out_ref[:, :nh] = 0.5 * acc0 * (
            1.0 + jnp.tanh(c * (acc0 + 0.044715 * acc0 * acc0 * acc0)))
        out_ref[:, nh:] = 0.5 * acc1 * (
            1.0 + jnp.tanh(c * (acc1 + 0.044715 * acc1 * acc1 * acc1)))

        for rdma in sends:
            rdma.wait_send()

    return pl.pallas_call(
        body,
        out_shape=jax.ShapeDtypeStruct((mb, n), jnp.float32),
        in_specs=[
            pl.BlockSpec(memory_space=pltpu.VMEM),
            pl.BlockSpec(memory_space=pltpu.VMEM),
        ],
        out_specs=pl.BlockSpec(memory_space=pltpu.VMEM),
        scratch_shapes=[
            pltpu.VMEM((N_DEV, mb, kb), jnp.bfloat16),
            pltpu.VMEM((N_DEV, mb, kb), jnp.bfloat16),
            pltpu.SemaphoreType.DMA((N_DEV - 1,)),
            pltpu.SemaphoreType.DMA((N_DEV,)),
        ],
        compiler_params=pltpu.CompilerParams(collective_id=0),
    )(x, w_mat)
